# baseline (speedup 1.0000x reference)
import functools

import jax
import jax.numpy as jnp
from jax import lax
from jax.experimental import pallas as pl
from jax.experimental.pallas import tpu as pltpu
from jax.experimental.pallas import tpu_sc as plsc

_B, _C, _T = 16, 256, 1024
_K, _E, _G = 1024, 128, 2
_TT = 1024
_P = _T // _TT

_NW = 32
_EQ = 32
_NQ = _E // _EQ
_NPAIR = _B * _G
_PAIRS_PER_W = _NPAIR // (_NW // _NQ)


def _argmin_body(x_ref, cb_ref, idx_ref):
    xb = x_ref[0]
    cb = cb_ref[...]
    e2 = jnp.sum(cb * cb, axis=1)
    x2 = jnp.sum(xb * xb, axis=0)
    xe = lax.dot_general(cb, xb, (((1,), (0,)), ((), ())),
                         preferred_element_type=jnp.float32)
    s = (x2[None, :] + e2[:, None]) - 2.0 * xe
    m = jnp.min(s, axis=0)
    kio = lax.broadcasted_iota(jnp.int32, (_K, _TT), 0)
    idx_ref[0, 0, 0] = jnp.min(
        jnp.where(s == m[None, :], kio, jnp.int32(_K)), axis=0)


def _tc_argmin(xin, codebook):
    grid = (_B, _G, _P)
    return pl.pallas_call(
        _argmin_body,
        grid=grid,
        in_specs=[
            pl.BlockSpec((1, _E, _TT), lambda b, g, p: (b, 0, g * _P + p)),
            pl.BlockSpec((_K, _E), lambda b, g, p: (0, 0)),
        ],
        out_specs=pl.BlockSpec((1, 1, 1, _TT), lambda b, g, p: (b, g, 0, p)),
        out_shape=jax.ShapeDtypeStruct((_B, _G, 1, _T), jnp.int32),
        compiler_params=pltpu.CompilerParams(
            dimension_semantics=("parallel", "parallel", "parallel"),
        ),
    )(xin, codebook)


def _sc_gather_body(cbq_hbm, idx_hbm, out_hbm, cb_v, idx_v, out_v, sem):
    wid = lax.axis_index("s") * 2 + lax.axis_index("c")
    q = wid % _NQ
    pg = wid // _NQ
    pltpu.sync_copy(cbq_hbm.at[q], cb_v)
    for i in range(_PAIRS_PER_W):
        p = pg + 8 * i
        pltpu.sync_copy(idx_hbm.at[p], idx_v)

        def jbody(j, carry):
            rows = idx_v[pl.ds(j * 16, 16)]
            scaled = rows * _EQ
            for e in range(_EQ):
                out_v[e, pl.ds(j * 16, 16)] = plsc.load_gather(
                    cb_v, [scaled + e])
            return carry

        lax.fori_loop(0, _T // 16, jbody, 0)
        pltpu.sync_copy(out_v, out_hbm.at[p, pl.ds(q * _EQ, _EQ), :])


def _sc_gather(cbq, idx2d):
    mesh = plsc.VectorSubcoreMesh(core_axis_name="c", subcore_axis_name="s")
    k = functools.partial(
        pl.kernel,
        mesh=mesh,
        out_type=jax.ShapeDtypeStruct((_NPAIR, _E, _T), jnp.float32),
        scratch_types=[
            pltpu.VMEM((_K * _EQ,), jnp.float32),
            pltpu.VMEM((_T,), jnp.int32),
            pltpu.VMEM((_EQ, _T), jnp.float32),
            pltpu.SemaphoreType.DMA,
        ],
        compiler_params=pltpu.CompilerParams(needs_layout_passes=False),
    )(_sc_gather_body)
    return k(cbq, idx2d)


def kernel(x, codebook):
    xin = x.reshape(_B, _E, _G * _T)
    idx = _tc_argmin(xin, codebook)
    cbq = codebook.reshape(_K, _NQ, _EQ).transpose(1, 0, 2).reshape(
        _NQ, _K * _EQ)
    q = _sc_gather(cbq, idx.reshape(_NPAIR, _T))
    quantized = q.reshape(_B, _C, _T)
    indexes = idx.reshape(_B, _G, _T).transpose(1, 0, 2)
    return quantized, indexes


# speedup vs baseline: 1.3050x; 1.3050x over previous
import functools

import jax
import jax.numpy as jnp
from jax import lax
from jax.experimental import pallas as pl
from jax.experimental.pallas import tpu as pltpu
from jax.experimental.pallas import tpu_sc as plsc

_B, _C, _T = 16, 256, 1024
_K, _E, _G = 1024, 128, 2
_TT = 1024
_P = _T // _TT

_NW = 32
_EQ = 32
_NQ = _E // _EQ
_NPAIR = _B * _G
_PAIRS_PER_W = _NPAIR // (_NW // _NQ)


def _argmin_body(x_ref, cb_ref, idx_ref):
    xb = x_ref[0]
    cb = cb_ref[...]
    e2 = jnp.sum(cb * cb, axis=1)
    x2 = jnp.sum(xb * xb, axis=0)
    xe = lax.dot_general(cb, xb, (((1,), (0,)), ((), ())),
                         preferred_element_type=jnp.float32)
    s = (x2[None, :] + e2[:, None]) - 2.0 * xe
    m = jnp.min(s, axis=0)
    kio = lax.broadcasted_iota(jnp.int32, (_K, _TT), 0)
    idx_ref[0, 0, 0] = jnp.min(
        jnp.where(s == m[None, :], kio, jnp.int32(_K)), axis=0)


def _tc_argmin(xin, codebook):
    grid = (_B, _G, _P)
    return pl.pallas_call(
        _argmin_body,
        grid=grid,
        in_specs=[
            pl.BlockSpec((1, _E, _TT), lambda b, g, p: (b, 0, g * _P + p)),
            pl.BlockSpec((_K, _E), lambda b, g, p: (0, 0)),
        ],
        out_specs=pl.BlockSpec((1, 1, 1, _TT), lambda b, g, p: (b, g, 0, p)),
        out_shape=jax.ShapeDtypeStruct((_B, _G, 1, _T), jnp.int32),
        compiler_params=pltpu.CompilerParams(
            dimension_semantics=("parallel", "parallel", "parallel"),
        ),
    )(xin, codebook)


def _sc_gather_body(cbq_hbm, idx_hbm, out_hbm, cb_v, idx_v, scaled_v, out_v,
                    sem):
    wid = lax.axis_index("s") * 2 + lax.axis_index("c")
    q = wid % _NQ
    pg = wid // _NQ
    pltpu.sync_copy(cbq_hbm.at[q], cb_v)
    for i in range(_PAIRS_PER_W):
        p = pg + 8 * i
        pltpu.sync_copy(idx_hbm.at[p], idx_v)

        @plsc.parallel_loop(0, _T // 16, unroll=4)
        def _scale(j):
            scaled_v[pl.ds(j * 16, 16)] = idx_v[pl.ds(j * 16, 16)] * _EQ

        @plsc.parallel_loop(0, (_T // 16) * _EQ, unroll=8)
        def _gather(i2):
            j = lax.shift_right_logical(i2, 5)
            e = lax.bitwise_and(i2, _EQ - 1)
            fidx = scaled_v[pl.ds(j * 16, 16)] + e
            out_v[e, pl.ds(j * 16, 16)] = plsc.load_gather(cb_v, [fidx])

        pltpu.sync_copy(out_v, out_hbm.at[p, pl.ds(q * _EQ, _EQ), :])


def _sc_gather(cbq, idx2d):
    mesh = plsc.VectorSubcoreMesh(core_axis_name="c", subcore_axis_name="s")
    k = functools.partial(
        pl.kernel,
        mesh=mesh,
        out_type=jax.ShapeDtypeStruct((_NPAIR, _E, _T), jnp.float32),
        scratch_types=[
            pltpu.VMEM((_K * _EQ,), jnp.float32),
            pltpu.VMEM((_T,), jnp.int32),
            pltpu.VMEM((_T,), jnp.int32),
            pltpu.VMEM((_EQ, _T), jnp.float32),
            pltpu.SemaphoreType.DMA,
        ],
        compiler_params=pltpu.CompilerParams(needs_layout_passes=False),
    )(_sc_gather_body)
    return k(cbq, idx2d)


def kernel(x, codebook):
    xin = x.reshape(_B, _E, _G * _T)
    idx = _tc_argmin(xin, codebook)
    cbq = codebook.reshape(_K, _NQ, _EQ).transpose(1, 0, 2).reshape(
        _NQ, _K * _EQ)
    q = _sc_gather(cbq, idx.reshape(_NPAIR, _T))
    quantized = q.reshape(_B, _C, _T)
    indexes = idx.reshape(_B, _G, _T).transpose(1, 0, 2)
    return quantized, indexes


# speedup vs baseline: 1.3193x; 1.0109x over previous
import functools

import jax
import jax.numpy as jnp
from jax import lax
from jax.experimental import pallas as pl
from jax.experimental.pallas import tpu as pltpu
from jax.experimental.pallas import tpu_sc as plsc

_B, _C, _T = 16, 256, 1024
_K, _E, _G = 1024, 128, 2
_TT = 1024
_P = _T // _TT

_NW = 32
_EQ = 32
_NQ = _E // _EQ
_NPAIR = _B * _G
_PAIRS_PER_W = _NPAIR // (_NW // _NQ)


def _argmin_body(x_ref, cb_ref, idx_ref):
    xb = x_ref[0]
    cb = cb_ref[...]
    e2 = jnp.sum(cb * cb, axis=1)
    x2 = jnp.sum(xb * xb, axis=0)
    xe = lax.dot_general(cb, xb, (((1,), (0,)), ((), ())),
                         preferred_element_type=jnp.float32)
    s = (x2[None, :] + e2[:, None]) - 2.0 * xe
    m = jnp.min(s, axis=0)
    kio = lax.broadcasted_iota(jnp.int32, (_K, _TT), 0)
    idx_ref[0, 0, 0] = jnp.min(
        jnp.where(s == m[None, :], kio, jnp.int32(_K)), axis=0)


def _tc_argmin(xin, codebook):
    grid = (_B, _G, _P)
    return pl.pallas_call(
        _argmin_body,
        grid=grid,
        in_specs=[
            pl.BlockSpec((1, _E, _TT), lambda b, g, p: (b, 0, g * _P + p)),
            pl.BlockSpec((_K, _E), lambda b, g, p: (0, 0)),
        ],
        out_specs=pl.BlockSpec((1, 1, 1, _TT), lambda b, g, p: (b, g, 0, p)),
        out_shape=jax.ShapeDtypeStruct((_B, _G, 1, _T), jnp.int32),
        compiler_params=pltpu.CompilerParams(
            dimension_semantics=("parallel", "parallel", "parallel"),
        ),
    )(xin, codebook)


def _sc_gather_body(cbq_hbm, idx_hbm, out_hbm, cb_v, idx_v, scaled_v, out_v,
                    sem):
    wid = lax.axis_index("s") * 2 + lax.axis_index("c")
    q = wid % _NQ
    pg = wid // _NQ
    pltpu.sync_copy(cbq_hbm.at[q], cb_v)
    copies = [None, None]
    for i in range(_PAIRS_PER_W):
        p = pg + 8 * i
        buf = i % 2
        pltpu.sync_copy(idx_hbm.at[p], idx_v)

        @plsc.parallel_loop(0, _T // 16, unroll=4)
        def _scale(j):
            scaled_v[pl.ds(j * 16, 16)] = idx_v[pl.ds(j * 16, 16)] * _EQ

        if copies[buf] is not None:
            copies[buf].wait()

        @plsc.parallel_loop(0, (_T // 16) * _EQ, unroll=16)
        def _gather(i2):
            j = lax.shift_right_logical(i2, 5)
            e = lax.bitwise_and(i2, _EQ - 1)
            fidx = scaled_v[pl.ds(j * 16, 16)] + e
            out_v[buf, e, pl.ds(j * 16, 16)] = plsc.load_gather(cb_v, [fidx])

        copies[buf] = pltpu.async_copy(
            out_v.at[buf], out_hbm.at[p, pl.ds(q * _EQ, _EQ), :], sem)
    for c in copies:
        c.wait()


def _sc_gather(cbq, idx2d):
    mesh = plsc.VectorSubcoreMesh(core_axis_name="c", subcore_axis_name="s")
    k = functools.partial(
        pl.kernel,
        mesh=mesh,
        out_type=jax.ShapeDtypeStruct((_NPAIR, _E, _T), jnp.float32),
        scratch_types=[
            pltpu.VMEM((_K * _EQ,), jnp.float32),
            pltpu.VMEM((_T,), jnp.int32),
            pltpu.VMEM((_T,), jnp.int32),
            pltpu.VMEM((2, _EQ, _T), jnp.float32),
            pltpu.SemaphoreType.DMA,
        ],
        compiler_params=pltpu.CompilerParams(needs_layout_passes=False),
    )(_sc_gather_body)
    return k(cbq, idx2d)


def kernel(x, codebook):
    xin = x.reshape(_B, _E, _G * _T)
    idx = _tc_argmin(xin, codebook)
    cbq = codebook.reshape(_K, _NQ, _EQ).transpose(1, 0, 2).reshape(
        _NQ, _K * _EQ)
    q = _sc_gather(cbq, idx.reshape(_NPAIR, _T))
    quantized = q.reshape(_B, _C, _T)
    indexes = idx.reshape(_B, _G, _T).transpose(1, 0, 2)
    return quantized, indexes


# speedup vs baseline: 1.8173x; 1.3775x over previous
import functools

import jax
import jax.numpy as jnp
from jax import lax
from jax.experimental import pallas as pl
from jax.experimental.pallas import tpu as pltpu
from jax.experimental.pallas import tpu_sc as plsc

_B, _C, _T = 16, 256, 1024
_K, _E, _G = 1024, 128, 2
_TT = 1024
_P = _T // _TT

_NW = 32
_EQ = 32
_NQ = _E // _EQ
_NPAIR = _B * _G
_PAIRS_PER_W = _NPAIR // (_NW // _NQ)


def _argmin_body(x_ref, cb_ref, idx_ref):
    xb = x_ref[0]
    cb = cb_ref[...]
    e2 = jnp.sum(cb * cb, axis=1)
    x2 = jnp.sum(xb * xb, axis=0)
    xe = lax.dot_general(cb, xb, (((1,), (0,)), ((), ())),
                         preferred_element_type=jnp.float32)
    s = (x2[None, :] + e2[:, None]) - 2.0 * xe
    m = jnp.min(s, axis=0)
    kio = lax.broadcasted_iota(jnp.int32, (_K, _TT), 0)
    idx_ref[0, 0, 0] = jnp.min(
        jnp.where(s == m[None, :], kio, jnp.int32(_K)), axis=0)


def _tc_argmin(xin, codebook):
    grid = (_B, _G, _P)
    return pl.pallas_call(
        _argmin_body,
        grid=grid,
        in_specs=[
            pl.BlockSpec((1, _E, _TT), lambda b, g, p: (b, 0, g * _P + p)),
            pl.BlockSpec((_K, _E), lambda b, g, p: (0, 0)),
        ],
        out_specs=pl.BlockSpec((1, 1, 1, _TT), lambda b, g, p: (b, g, 0, p)),
        out_shape=jax.ShapeDtypeStruct((_B, _G, 1, _T), jnp.int32),
        compiler_params=pltpu.CompilerParams(
            dimension_semantics=("parallel", "parallel", "parallel"),
        ),
    )(xin, codebook)


def _sc_gather_body(cbq_hbm, idx_hbm, out_hbm, cb_v, idx_v, out_v, sem):
    wid = lax.axis_index("s") * 2 + lax.axis_index("c")
    q = wid % _NQ
    pg = wid // _NQ
    pltpu.sync_copy(cbq_hbm.at[q], cb_v)
    copies = [None, None]
    for i in range(_PAIRS_PER_W):
        p = pg + 8 * i
        buf = i % 2
        pltpu.sync_copy(idx_hbm.at[p], idx_v)

        if copies[buf] is not None:
            copies[buf].wait()

        @plsc.parallel_loop(0, (_T // 16) * _EQ, unroll=16)
        def _gather(i2):
            j = lax.shift_right_logical(i2, 5)
            e = lax.bitwise_and(i2, _EQ - 1)
            fidx = idx_v[pl.ds(j * 16, 16)] + lax.shift_left(e, 10)
            out_v[buf, e, pl.ds(j * 16, 16)] = plsc.load_gather(cb_v, [fidx])

        copies[buf] = pltpu.async_copy(
            out_v.at[buf], out_hbm.at[p, pl.ds(q * _EQ, _EQ), :], sem)
    for c in copies:
        c.wait()


def _sc_gather(cbq, idx2d):
    mesh = plsc.VectorSubcoreMesh(core_axis_name="c", subcore_axis_name="s")
    k = functools.partial(
        pl.kernel,
        mesh=mesh,
        out_type=jax.ShapeDtypeStruct((_NPAIR, _E, _T), jnp.float32),
        scratch_types=[
            pltpu.VMEM((_K * _EQ,), jnp.float32),
            pltpu.VMEM((_T,), jnp.int32),
            pltpu.VMEM((2, _EQ, _T), jnp.float32),
            pltpu.SemaphoreType.DMA,
        ],
        compiler_params=pltpu.CompilerParams(needs_layout_passes=False),
    )(_sc_gather_body)
    return k(cbq, idx2d)


def kernel(x, codebook):
    xin = x.reshape(_B, _E, _G * _T)
    idx = _tc_argmin(xin, codebook)
    cbq = codebook.T.reshape(_NQ, _EQ * _K)
    q = _sc_gather(cbq, idx.reshape(_NPAIR, _T))
    quantized = q.reshape(_B, _C, _T)
    indexes = idx.reshape(_B, _G, _T).transpose(1, 0, 2)
    return quantized, indexes
